# Initial kernel scaffold; baseline (speedup 1.0000x reference)
import jax
import jax.numpy as jnp
from jax.experimental import pallas as pl


def kernel(ref_points_c, src_points_c, gt_node_corr_overlaps, gt_node_corr_indices, ref_node_corr_indices, src_node_corr_indices, ref_corr_points, src_corr_points, transform, estimated_transform):
    raise NotImplementedError("write your pallas kernel here")


# speedup vs baseline: 1.8598x; 1.8598x over previous
import functools

import jax
import jax.numpy as jnp
import numpy as np
from jax.experimental import pallas as pl
from jax.experimental.pallas import tpu as pltpu

_N_FINE = 100000
_N_FINE_PAD = 100096
_LANES_F = _N_FINE_PAD // 8


def _euler_atan2_args(t_ref, et_ref, acc_num, acc_den):
    lane = jax.lax.broadcasted_iota(jnp.int32, (1, 128), 1)
    for base, ref in ((0, t_ref), (3, et_ref)):
        r00 = ref[0, 0]
        r10 = ref[1, 0]
        r20 = ref[2, 0]
        r21 = ref[2, 1]
        r22 = ref[2, 2]
        sy = jnp.sqrt(r00 * r00 + r10 * r10)
        acc_num = jnp.where(lane == base + 0, r21, acc_num)
        acc_den = jnp.where(lane == base + 0, r22, acc_den)
        acc_num = jnp.where(lane == base + 1, -r20, acc_num)
        acc_den = jnp.where(lane == base + 1, sy, acc_den)
        acc_num = jnp.where(lane == base + 2, r10, acc_num)
        acc_den = jnp.where(lane == base + 2, r00, acc_den)
    return acc_num, acc_den


def _body(ref_f, src_f, gtr_ref, gts_ref, ov_ref, qr_ref, qs_ref,
          t_ref, et_ref, out_ref, g_scr, q_scr):
    rx = ref_f[0:8, :]
    ry = ref_f[8:16, :]
    rz = ref_f[16:24, :]
    sx = src_f[0:8, :]
    sy_ = src_f[8:16, :]
    sz = src_f[16:24, :]
    dx = rx - (t_ref[0, 0] * sx + t_ref[0, 1] * sy_ + t_ref[0, 2] * sz + t_ref[0, 3])
    dy = ry - (t_ref[1, 0] * sx + t_ref[1, 1] * sy_ + t_ref[1, 2] * sz + t_ref[1, 3])
    dz = rz - (t_ref[2, 0] * sx + t_ref[2, 1] * sy_ + t_ref[2, 2] * sz + t_ref[2, 3])
    d2 = dx * dx + dy * dy + dz * dz
    f_count = jnp.sum(jnp.where(d2 < 0.01, 1.0, 0.0))
    f_precision = f_count * (1.0 / _N_FINE)

    g_scr[...] = jnp.where(ov_ref[...] > 0.0,
                           gtr_ref[...] * 2048 + gts_ref[...],
                           jnp.full_like(gtr_ref[...], -1))
    q_scr[...] = qr_ref[...] * 2048 + qs_ref[...]

    def row_body(i, cntvec):
        qrow = q_scr[pl.ds(i, 1), :]

        def chunk_body(c, acc):
            gc = g_scr[pl.ds(c * 1024, 1024), :]
            eq = (gc == qrow).astype(jnp.int32)
            return jnp.maximum(acc, jnp.max(eq, axis=0, keepdims=True))

        acc = jax.lax.fori_loop(0, 8, chunk_body, jnp.zeros((1, 128), jnp.int32),
                                unroll=True)
        return cntvec + acc.astype(jnp.float32)

    cntvec = jax.lax.fori_loop(0, 32, row_body, jnp.zeros((1, 128), jnp.float32))
    c_precision = jnp.sum(cntvec) * (1.0 / 4096.0)

    num, den = _euler_atan2_args(t_ref, et_ref,
                                 jnp.zeros((1, 128), jnp.float32),
                                 jnp.ones((1, 128), jnp.float32))
    e = jnp.arctan2(num, den) * np.float32(180.0 / np.pi)
    lane = jax.lax.broadcasted_iota(jnp.int32, (1, 128), 1)
    e_est_shift = jnp.where(lane < 3, jnp.roll(e, -3, axis=1), 0.0)
    d_e = jnp.where(lane < 3, e - e_est_shift, 0.0)
    r_mse = jnp.sum(d_e * d_e) * (1.0 / 3.0)
    r_mae = jnp.sum(jnp.abs(d_e)) * (1.0 / 3.0)
    dt0 = t_ref[0, 3] - et_ref[0, 3]
    dt1 = t_ref[1, 3] - et_ref[1, 3]
    dt2 = t_ref[2, 3] - et_ref[2, 3]
    t_mse = (dt0 * dt0 + dt1 * dt1 + dt2 * dt2) * (1.0 / 3.0)
    t_mae = (jnp.abs(dt0) + jnp.abs(dt1) + jnp.abs(dt2)) * (1.0 / 3.0)

    out = jnp.zeros((1, 128), jnp.float32)
    out = jnp.where(lane == 0, c_precision, out)
    out = jnp.where(lane == 1, f_precision, out)
    out = jnp.where(lane == 2, r_mse, out)
    out = jnp.where(lane == 3, r_mae, out)
    out = jnp.where(lane == 4, t_mse, out)
    out = jnp.where(lane == 5, t_mae, out)
    out_ref[...] = out


@functools.partial(jax.jit, static_argnames=())
def kernel(ref_points_c, src_points_c, gt_node_corr_overlaps,
           gt_node_corr_indices, ref_node_corr_indices, src_node_corr_indices,
           ref_corr_points, src_corr_points, transform, estimated_transform):
    del ref_points_c, src_points_c

    pad = _N_FINE_PAD - _N_FINE
    ref_f = jnp.pad(ref_corr_points.T, ((0, 0), (0, pad)),
                    constant_values=1e6).reshape(24, _LANES_F)
    src_f = jnp.pad(src_corr_points.T, ((0, 0), (0, pad)),
                    constant_values=0.0).reshape(24, _LANES_F)
    gtr = gt_node_corr_indices[:, 0].astype(jnp.int32).reshape(8192, 1)
    gts = gt_node_corr_indices[:, 1].astype(jnp.int32).reshape(8192, 1)
    ov = gt_node_corr_overlaps.reshape(8192, 1)
    qr = ref_node_corr_indices.astype(jnp.int32).reshape(32, 128)
    qs = src_node_corr_indices.astype(jnp.int32).reshape(32, 128)

    smem_spec = pl.BlockSpec(memory_space=pltpu.SMEM)
    vmem_spec = pl.BlockSpec(memory_space=pltpu.VMEM)
    out = pl.pallas_call(
        _body,
        out_shape=jax.ShapeDtypeStruct((1, 128), jnp.float32),
        in_specs=[vmem_spec] * 7 + [smem_spec, smem_spec],
        out_specs=vmem_spec,
        scratch_shapes=[pltpu.VMEM((8192, 1), jnp.int32),
                        pltpu.VMEM((32, 128), jnp.int32)],
    )(ref_f, src_f, gtr, gts, ov, qr, qs, transform, estimated_transform)
    return out[0, 0:6]


# speedup vs baseline: 2.7942x; 1.5024x over previous
import functools

import jax
import jax.numpy as jnp
import numpy as np
from jax import lax
from jax.experimental import pallas as pl
from jax.experimental.pallas import tpu as pltpu
from jax.experimental.pallas import tpu_sc as plsc

_N_FINE = 100000
_N_FINE_PAD = 100096
_LANES_F = _N_FINE_PAD // 8

_NQ = 4096
_NG = 8192
_NW = 32
_WPT = 65536


def _sc_body(qr_h, qs_h, gtr_h, gts_h, ov_h, cnt_h,
             qr_v, qs_v, gr_v, gs_v, ov_v, map_v, acc_v):
    cid = lax.axis_index("c")
    sid = lax.axis_index("s")
    wid = sid * 2 + cid
    pltpu.sync_copy(qr_h, qr_v)
    pltpu.sync_copy(qs_h, qs_v)
    pltpu.sync_copy(gtr_h, gr_v)
    pltpu.sync_copy(gts_h, gs_v)
    pltpu.sync_copy(ov_h, ov_v)

    zeros16 = jnp.zeros((16,), jnp.int32)
    one16 = jnp.full((16,), 1, jnp.int32)

    def _codes(rv, sv, i):
        r = rv[pl.ds(i * 16, 16)]
        s = sv[pl.ds(i * 16, 16)]
        code = r * 2048 + s
        w = lax.shift_right_logical(code, 1)
        own = lax.shift_right_logical(w, 16) == wid
        local = jnp.bitwise_and(w, 65535)
        return code, own, local

    def _zero_q(i, carry):
        _, own, local = _codes(qr_v, qs_v, i)
        plsc.store_scatter(map_v, [local], zeros16, mask=own)
        return carry

    lax.fori_loop(0, _NQ // 16, _zero_q, 0)

    def _add_gt(i, carry):
        code, own, local = _codes(gr_v, gs_v, i)
        o = ov_v[pl.ds(i * 16, 16)]
        mask = jnp.logical_and(own, o > 0.0)
        val = lax.shift_left(one16, lax.shift_left(jnp.bitwise_and(code, 1), 4))
        plsc.addupdate_scatter(map_v, [local], val, mask=mask)
        return carry

    lax.fori_loop(0, _NG // 16, _add_gt, 0)

    def _gather_q(i, acc):
        code, own, local = _codes(qr_v, qs_v, i)
        v = plsc.load_gather(map_v, [local], mask=own)
        half = jnp.bitwise_and(
            lax.shift_right_logical(v, lax.shift_left(jnp.bitwise_and(code, 1), 4)),
            65535)
        hit = jnp.logical_and(own, half > 0)
        return acc + jnp.where(hit, 1.0, 0.0).astype(jnp.float32)

    acc = lax.fori_loop(0, _NQ // 16, _gather_q, jnp.zeros((16,), jnp.float32))
    acc_v[...] = acc
    pltpu.sync_copy(acc_v, cnt_h.at[pl.ds(wid * 16, 16)])


_sc_coarse = functools.partial(
    pl.kernel,
    out_type=jax.ShapeDtypeStruct((_NW * 16,), jnp.float32),
    mesh=plsc.VectorSubcoreMesh(core_axis_name="c", subcore_axis_name="s"),
    compiler_params=pltpu.CompilerParams(needs_layout_passes=False),
    scratch_types=[
        pltpu.VMEM((_NQ,), jnp.int32),
        pltpu.VMEM((_NQ,), jnp.int32),
        pltpu.VMEM((_NG,), jnp.int32),
        pltpu.VMEM((_NG,), jnp.int32),
        pltpu.VMEM((_NG,), jnp.float32),
        pltpu.VMEM((_WPT,), jnp.int32),
        pltpu.VMEM((16,), jnp.float32),
    ],
)(_sc_body)


def _euler_atan2_args(t_ref, et_ref, acc_num, acc_den):
    lane = jax.lax.broadcasted_iota(jnp.int32, (1, 128), 1)
    for base, ref in ((0, t_ref), (3, et_ref)):
        r00 = ref[0, 0]
        r10 = ref[1, 0]
        r20 = ref[2, 0]
        r21 = ref[2, 1]
        r22 = ref[2, 2]
        sy = jnp.sqrt(r00 * r00 + r10 * r10)
        acc_num = jnp.where(lane == base + 0, r21, acc_num)
        acc_den = jnp.where(lane == base + 0, r22, acc_den)
        acc_num = jnp.where(lane == base + 1, -r20, acc_num)
        acc_den = jnp.where(lane == base + 1, sy, acc_den)
        acc_num = jnp.where(lane == base + 2, r10, acc_num)
        acc_den = jnp.where(lane == base + 2, r00, acc_den)
    return acc_num, acc_den


def _tc_body(ref_f, src_f, cc_ref, t_ref, et_ref, out_ref):
    rx = ref_f[0:8, :]
    ry = ref_f[8:16, :]
    rz = ref_f[16:24, :]
    sx = src_f[0:8, :]
    sy_ = src_f[8:16, :]
    sz = src_f[16:24, :]
    dx = rx - (t_ref[0, 0] * sx + t_ref[0, 1] * sy_ + t_ref[0, 2] * sz + t_ref[0, 3])
    dy = ry - (t_ref[1, 0] * sx + t_ref[1, 1] * sy_ + t_ref[1, 2] * sz + t_ref[1, 3])
    dz = rz - (t_ref[2, 0] * sx + t_ref[2, 1] * sy_ + t_ref[2, 2] * sz + t_ref[2, 3])
    d2 = dx * dx + dy * dy + dz * dz
    f_count = jnp.sum(jnp.where(d2 < 0.01, 1.0, 0.0))
    f_precision = f_count * (1.0 / _N_FINE)

    c_precision = jnp.sum(cc_ref[...]) * (1.0 / _NQ)

    num, den = _euler_atan2_args(t_ref, et_ref,
                                 jnp.zeros((1, 128), jnp.float32),
                                 jnp.ones((1, 128), jnp.float32))
    e = jnp.arctan2(num, den) * np.float32(180.0 / np.pi)
    lane = jax.lax.broadcasted_iota(jnp.int32, (1, 128), 1)
    e_est_shift = jnp.where(lane < 3, jnp.roll(e, -3, axis=1), 0.0)
    d_e = jnp.where(lane < 3, e - e_est_shift, 0.0)
    r_mse = jnp.sum(d_e * d_e) * (1.0 / 3.0)
    r_mae = jnp.sum(jnp.abs(d_e)) * (1.0 / 3.0)
    dt0 = t_ref[0, 3] - et_ref[0, 3]
    dt1 = t_ref[1, 3] - et_ref[1, 3]
    dt2 = t_ref[2, 3] - et_ref[2, 3]
    t_mse = (dt0 * dt0 + dt1 * dt1 + dt2 * dt2) * (1.0 / 3.0)
    t_mae = (jnp.abs(dt0) + jnp.abs(dt1) + jnp.abs(dt2)) * (1.0 / 3.0)

    out = jnp.zeros((1, 128), jnp.float32)
    out = jnp.where(lane == 0, c_precision, out)
    out = jnp.where(lane == 1, f_precision, out)
    out = jnp.where(lane == 2, r_mse, out)
    out = jnp.where(lane == 3, r_mae, out)
    out = jnp.where(lane == 4, t_mse, out)
    out = jnp.where(lane == 5, t_mae, out)
    out_ref[...] = out


def kernel(ref_points_c, src_points_c, gt_node_corr_overlaps,
           gt_node_corr_indices, ref_node_corr_indices, src_node_corr_indices,
           ref_corr_points, src_corr_points, transform, estimated_transform):
    del ref_points_c, src_points_c

    cnt = _sc_coarse(
        ref_node_corr_indices.astype(jnp.int32),
        src_node_corr_indices.astype(jnp.int32),
        gt_node_corr_indices[:, 0].astype(jnp.int32),
        gt_node_corr_indices[:, 1].astype(jnp.int32),
        gt_node_corr_overlaps,
    )

    pad = _N_FINE_PAD - _N_FINE
    ref_f = jnp.pad(ref_corr_points.T, ((0, 0), (0, pad)),
                    constant_values=1e6).reshape(24, _LANES_F)
    src_f = jnp.pad(src_corr_points.T, ((0, 0), (0, pad)),
                    constant_values=0.0).reshape(24, _LANES_F)
    cc = cnt.reshape(4, 128)

    smem_spec = pl.BlockSpec(memory_space=pltpu.SMEM)
    vmem_spec = pl.BlockSpec(memory_space=pltpu.VMEM)
    out = pl.pallas_call(
        _tc_body,
        out_shape=jax.ShapeDtypeStruct((1, 128), jnp.float32),
        in_specs=[vmem_spec, vmem_spec, vmem_spec, smem_spec, smem_spec],
        out_specs=vmem_spec,
    )(ref_f, src_f, cc, transform, estimated_transform)
    return out[0, 0:6]


# speedup vs baseline: 3.5754x; 1.2796x over previous
import functools

import jax
import jax.numpy as jnp
import numpy as np
from jax import lax
from jax.experimental import pallas as pl
from jax.experimental.pallas import tpu as pltpu
from jax.experimental.pallas import tpu_sc as plsc

_N_FINE = 100000
_N_FINE_PAD = 100096
_LANES_F = _N_FINE_PAD // 8

_NQ = 4096
_NG = 8192
_NW = 32
_WPT = 65536


def _sc_body(qr_h, qs_h, gtr_h, gts_h, ov_h, cnt_h,
             qr_v, qs_v, gr_v, gs_v, ov_v, map_v, acc_v,
             sem0, sem1, sem2, sem3, sem4):
    cid = lax.axis_index("c")
    sid = lax.axis_index("s")
    wid = sid * 2 + cid
    cqr = pltpu.async_copy(qr_h, qr_v, sem0)
    cqs = pltpu.async_copy(qs_h, qs_v, sem1)
    cgr = pltpu.async_copy(gtr_h, gr_v, sem2)
    cgs = pltpu.async_copy(gts_h, gs_v, sem3)
    cov = pltpu.async_copy(ov_h, ov_v, sem4)

    zeros16 = jnp.zeros((16,), jnp.int32)
    one16 = jnp.full((16,), 1, jnp.int32)

    def _codes(rv, sv, i):
        r = rv[pl.ds(i, 16)]
        s = sv[pl.ds(i, 16)]
        code = r * 2048 + s
        own = lax.shift_right_logical(code, 17) == wid
        local = jnp.bitwise_and(lax.shift_right_logical(code, 1), 65535)
        return code, own, local

    cqr.wait()
    cqs.wait()

    @plsc.parallel_loop(0, _NQ, step=16, unroll=4)
    def _zero_q(i):
        _, own, local = _codes(qr_v, qs_v, i)
        plsc.store_scatter(map_v, [local], zeros16, mask=own)

    cgr.wait()
    cgs.wait()
    cov.wait()

    @plsc.parallel_loop(0, _NG, step=16, unroll=4)
    def _add_gt(i):
        code, own, local = _codes(gr_v, gs_v, i)
        o = ov_v[pl.ds(i, 16)]
        mask = jnp.logical_and(own, o > 0.0)
        val = lax.shift_left(one16, lax.shift_left(jnp.bitwise_and(code, 1), 4))
        plsc.addupdate_scatter(map_v, [local], val, mask=mask)

    @plsc.parallel_loop(0, _NQ, step=16, unroll=4,
                        carry=jnp.zeros((16,), jnp.float32))
    def _gather_q(i, acc):
        code, own, local = _codes(qr_v, qs_v, i)
        v = plsc.load_gather(map_v, [local], mask=own)
        half = jnp.bitwise_and(
            lax.shift_right_logical(v, lax.shift_left(jnp.bitwise_and(code, 1), 4)),
            65535)
        hit = jnp.logical_and(own, half > 0)
        return acc + jnp.where(hit, 1.0, 0.0).astype(jnp.float32)

    acc_v[...] = _gather_q
    pltpu.sync_copy(acc_v, cnt_h.at[pl.ds(wid * 16, 16)])


_sc_coarse = functools.partial(
    pl.kernel,
    out_type=jax.ShapeDtypeStruct((_NW * 16,), jnp.float32),
    mesh=plsc.VectorSubcoreMesh(core_axis_name="c", subcore_axis_name="s"),
    compiler_params=pltpu.CompilerParams(needs_layout_passes=False),
    scratch_types=[
        pltpu.VMEM((_NQ,), jnp.int32),
        pltpu.VMEM((_NQ,), jnp.int32),
        pltpu.VMEM((_NG,), jnp.int32),
        pltpu.VMEM((_NG,), jnp.int32),
        pltpu.VMEM((_NG,), jnp.float32),
        pltpu.VMEM((_WPT,), jnp.int32),
        pltpu.VMEM((16,), jnp.float32),
        pltpu.SemaphoreType.DMA,
        pltpu.SemaphoreType.DMA,
        pltpu.SemaphoreType.DMA,
        pltpu.SemaphoreType.DMA,
        pltpu.SemaphoreType.DMA,
    ],
)(_sc_body)


def _euler_atan2_args(t_ref, et_ref, acc_num, acc_den):
    lane = jax.lax.broadcasted_iota(jnp.int32, (1, 128), 1)
    for base, ref in ((0, t_ref), (3, et_ref)):
        r00 = ref[0, 0]
        r10 = ref[1, 0]
        r20 = ref[2, 0]
        r21 = ref[2, 1]
        r22 = ref[2, 2]
        sy = jnp.sqrt(r00 * r00 + r10 * r10)
        acc_num = jnp.where(lane == base + 0, r21, acc_num)
        acc_den = jnp.where(lane == base + 0, r22, acc_den)
        acc_num = jnp.where(lane == base + 1, -r20, acc_num)
        acc_den = jnp.where(lane == base + 1, sy, acc_den)
        acc_num = jnp.where(lane == base + 2, r10, acc_num)
        acc_den = jnp.where(lane == base + 2, r00, acc_den)
    return acc_num, acc_den


def _tc_body(ref_f, src_f, cc_ref, t_ref, et_ref, out_ref):
    rx = ref_f[0:8, :]
    ry = ref_f[8:16, :]
    rz = ref_f[16:24, :]
    sx = src_f[0:8, :]
    sy_ = src_f[8:16, :]
    sz = src_f[16:24, :]
    dx = rx - (t_ref[0, 0] * sx + t_ref[0, 1] * sy_ + t_ref[0, 2] * sz + t_ref[0, 3])
    dy = ry - (t_ref[1, 0] * sx + t_ref[1, 1] * sy_ + t_ref[1, 2] * sz + t_ref[1, 3])
    dz = rz - (t_ref[2, 0] * sx + t_ref[2, 1] * sy_ + t_ref[2, 2] * sz + t_ref[2, 3])
    d2 = dx * dx + dy * dy + dz * dz
    f_count = jnp.sum(jnp.where(d2 < 0.01, 1.0, 0.0))
    f_precision = f_count * (1.0 / _N_FINE)

    c_precision = jnp.sum(cc_ref[...]) * (1.0 / _NQ)

    num, den = _euler_atan2_args(t_ref, et_ref,
                                 jnp.zeros((1, 128), jnp.float32),
                                 jnp.ones((1, 128), jnp.float32))
    e = jnp.arctan2(num, den) * np.float32(180.0 / np.pi)
    lane = jax.lax.broadcasted_iota(jnp.int32, (1, 128), 1)
    e_est_shift = jnp.where(lane < 3, jnp.roll(e, -3, axis=1), 0.0)
    d_e = jnp.where(lane < 3, e - e_est_shift, 0.0)
    r_mse = jnp.sum(d_e * d_e) * (1.0 / 3.0)
    r_mae = jnp.sum(jnp.abs(d_e)) * (1.0 / 3.0)
    dt0 = t_ref[0, 3] - et_ref[0, 3]
    dt1 = t_ref[1, 3] - et_ref[1, 3]
    dt2 = t_ref[2, 3] - et_ref[2, 3]
    t_mse = (dt0 * dt0 + dt1 * dt1 + dt2 * dt2) * (1.0 / 3.0)
    t_mae = (jnp.abs(dt0) + jnp.abs(dt1) + jnp.abs(dt2)) * (1.0 / 3.0)

    out = jnp.zeros((1, 128), jnp.float32)
    out = jnp.where(lane == 0, c_precision, out)
    out = jnp.where(lane == 1, f_precision, out)
    out = jnp.where(lane == 2, r_mse, out)
    out = jnp.where(lane == 3, r_mae, out)
    out = jnp.where(lane == 4, t_mse, out)
    out = jnp.where(lane == 5, t_mae, out)
    out_ref[...] = out


def kernel(ref_points_c, src_points_c, gt_node_corr_overlaps,
           gt_node_corr_indices, ref_node_corr_indices, src_node_corr_indices,
           ref_corr_points, src_corr_points, transform, estimated_transform):
    del ref_points_c, src_points_c

    cnt = _sc_coarse(
        ref_node_corr_indices.astype(jnp.int32),
        src_node_corr_indices.astype(jnp.int32),
        gt_node_corr_indices[:, 0].astype(jnp.int32),
        gt_node_corr_indices[:, 1].astype(jnp.int32),
        gt_node_corr_overlaps,
    )

    pad = _N_FINE_PAD - _N_FINE
    ref_f = jnp.pad(ref_corr_points.T, ((0, 0), (0, pad)),
                    constant_values=1e6).reshape(24, _LANES_F)
    src_f = jnp.pad(src_corr_points.T, ((0, 0), (0, pad)),
                    constant_values=0.0).reshape(24, _LANES_F)
    cc = cnt.reshape(4, 128)

    smem_spec = pl.BlockSpec(memory_space=pltpu.SMEM)
    vmem_spec = pl.BlockSpec(memory_space=pltpu.VMEM)
    out = pl.pallas_call(
        _tc_body,
        out_shape=jax.ShapeDtypeStruct((1, 128), jnp.float32),
        in_specs=[vmem_spec, vmem_spec, vmem_spec, smem_spec, smem_spec],
        out_specs=vmem_spec,
    )(ref_f, src_f, cc, transform, estimated_transform)
    return out[0, 0:6]
